# baseline (speedup 1.0000x reference)
import functools

import jax
import jax.numpy as jnp
from jax import lax
from jax.experimental import pallas as pl
from jax.experimental.pallas import tpu as pltpu
from jax.experimental.pallas import tpu_sc as plsc

_GRID = 10
_NC, _NS = 2, 16
_NW = _NC * _NS
_CHUNKS = 5


def _copy_x_body(xs_ref, xt_ref, u_ref, oxs_ref, oxt_ref, ou_ref):
    oxs_ref[...] = xs_ref[...]
    oxt_ref[...] = xt_ref[...]

    @pl.when(pl.program_id(0) == 0)
    def _():
        ou_ref[...] = u_ref[...]


def _sc_copy_body(rows_per_chunk, ea_hbm, out_hbm, buf):
    wid = lax.axis_index("s") * _NC + lax.axis_index("c")
    base = wid * rows_per_chunk * _CHUNKS

    def chunk(i, _):
        off = base + i * rows_per_chunk
        pltpu.sync_copy(ea_hbm.at[pl.ds(off, rows_per_chunk)], buf)
        pltpu.sync_copy(buf, out_hbm.at[pl.ds(off, rows_per_chunk)])
        return ()

    lax.fori_loop(0, _CHUNKS, chunk, ())


def kernel(x_s, x_t, edge_index, edge_attr, u, batch_e, batch_s, batch_t):
    del edge_index, batch_e, batch_s, batch_t
    n_s, d_feat = x_s.shape
    e, d_edge = edge_attr.shape
    bx = n_s // _GRID

    rows_per_chunk = e // (_NW * _CHUNKS)
    mesh = plsc.VectorSubcoreMesh(core_axis_name="c", subcore_axis_name="s")
    sc_copy = pl.kernel(
        functools.partial(_sc_copy_body, rows_per_chunk),
        out_type=jax.ShapeDtypeStruct(edge_attr.shape, edge_attr.dtype),
        mesh=mesh,
        scratch_types=[pltpu.VMEM((rows_per_chunk, d_edge), edge_attr.dtype)],
        compiler_params=pltpu.CompilerParams(use_tc_tiling_on_sc=True),
    )
    ea_o = sc_copy(edge_attr)

    xspecs = [
        pl.BlockSpec((bx, d_feat), lambda i: (i, 0)),
        pl.BlockSpec((bx, d_feat), lambda i: (i, 0)),
        pl.BlockSpec(u.shape, lambda i: (0, 0)),
    ]
    xs_o, xt_o, u_o = pl.pallas_call(
        _copy_x_body,
        grid=(_GRID,),
        in_specs=xspecs,
        out_specs=xspecs,
        out_shape=[
            jax.ShapeDtypeStruct(x_s.shape, x_s.dtype),
            jax.ShapeDtypeStruct(x_t.shape, x_t.dtype),
            jax.ShapeDtypeStruct(u.shape, u.dtype),
        ],
    )(x_s, x_t, u)

    return (xs_o, xt_o, ea_o, u_o)


# speedup vs baseline: 1.1555x; 1.1555x over previous
import jax
import jax.numpy as jnp
from jax.experimental import pallas as pl

_GRID = 10


def _copy_body(xs_ref, xt_ref, ea_ref, u_ref, oxs_ref, oxt_ref, oea_ref, ou_ref):
    oxs_ref[...] = xs_ref[...]
    oxt_ref[...] = xt_ref[...]
    oea_ref[...] = ea_ref[...]

    @pl.when(pl.program_id(0) == 0)
    def _():
        ou_ref[...] = u_ref[...]


def kernel(x_s, x_t, edge_index, edge_attr, u, batch_e, batch_s, batch_t):
    del edge_index, batch_e, batch_s, batch_t
    n_s, d_feat = x_s.shape
    e, d_edge = edge_attr.shape
    bx = n_s // _GRID
    be = e // _GRID

    specs = [
        pl.BlockSpec((bx, d_feat), lambda i: (i, 0)),
        pl.BlockSpec((bx, d_feat), lambda i: (i, 0)),
        pl.BlockSpec((be, d_edge), lambda i: (i, 0)),
        pl.BlockSpec(u.shape, lambda i: (0, 0)),
    ]
    outs = pl.pallas_call(
        _copy_body,
        grid=(_GRID,),
        in_specs=specs,
        out_specs=specs,
        out_shape=[
            jax.ShapeDtypeStruct(x_s.shape, x_s.dtype),
            jax.ShapeDtypeStruct(x_t.shape, x_t.dtype),
            jax.ShapeDtypeStruct(edge_attr.shape, edge_attr.dtype),
            jax.ShapeDtypeStruct(u.shape, u.dtype),
        ],
    )(x_s, x_t, edge_attr, u)
    return tuple(outs)
